# baseline (speedup 1.0000x reference)
import functools

import jax
import jax.numpy as jnp
from jax import lax
from jax.experimental import pallas as pl
from jax.experimental.pallas import tpu as pltpu
from jax.experimental.pallas import tpu_sc as plsc

N_NODES = 10000
N_EDGES = 320000
D = 128

NC = 2
NS = 16
NW = NC * NS

CHUNK = 128
CHUNKS_PER_TILE = 80
IDX_HALF = CHUNKS_PER_TILE // 2
E_PAD = NW * CHUNKS_PER_TILE * CHUNK
N_CHUNKS = E_PAD // CHUNK
N_PAD = 10240
ROWS_PER_TILE = N_PAD // NS

NBUF = 8
CHUNK_S = 32
CHUNKS_PER_TILE_S = E_PAD // (NW * CHUNK_S)
N_IDX_LOADS = 8
IDX_HALF_S = CHUNKS_PER_TILE_S // N_IDX_LOADS
N_CHUNKS_S = E_PAD // CHUNK_S


def _mesh():
    return plsc.VectorSubcoreMesh(
        core_axis_name="c", subcore_axis_name="s",
        num_cores=NC, num_subcores=NS)


def _sc_degree(dst_idx, ones_rows, zeros_rows):
    @functools.partial(
        pl.kernel,
        out_type=jax.ShapeDtypeStruct((NC, N_PAD, D), jnp.float32),
        mesh=_mesh(),
        scratch_types=[
            pltpu.VMEM((CHUNKS_PER_TILE, CHUNK), jnp.int32),
            pltpu.VMEM((CHUNK, D), jnp.float32),
            pltpu.VMEM_SHARED((N_PAD, D), jnp.float32),
        ],
    )
    def k(dst_hbm, ones_hbm, zero_hbm, out_hbm, dst_v, ones_v, acc_sh):
        c = lax.axis_index("c")
        s = lax.axis_index("s")
        wid = c * NS + s
        for kk in range(ROWS_PER_TILE // CHUNK):
            pltpu.sync_copy(
                zero_hbm, acc_sh.at[pl.ds(s * ROWS_PER_TILE + kk * CHUNK, CHUNK)])
        pltpu.sync_copy(dst_hbm.at[pl.ds(wid * CHUNKS_PER_TILE, CHUNKS_PER_TILE)], dst_v)
        pltpu.sync_copy(ones_hbm, ones_v)
        plsc.subcore_barrier()

        def body(j, carry):
            pltpu.sync_copy(ones_v, acc_sh.at[dst_v.at[j]], add=True)
            return carry

        lax.fori_loop(0, CHUNKS_PER_TILE, body, 0)
        plsc.subcore_barrier()
        pltpu.sync_copy(
            acc_sh.at[pl.ds(s * ROWS_PER_TILE, ROWS_PER_TILE)],
            out_hbm.at[c, pl.ds(s * ROWS_PER_TILE, ROWS_PER_TILE)])

    return k(dst_idx, ones_rows, zeros_rows)


def _sc_scatter(g, src_idx, dst_idx, zeros_rows):
    @functools.partial(
        pl.kernel,
        out_type=jax.ShapeDtypeStruct((NC, N_PAD, D), jnp.float32),
        mesh=_mesh(),
        scratch_types=[
            pltpu.VMEM((IDX_HALF_S, CHUNK_S), jnp.int32),
            pltpu.VMEM((IDX_HALF_S, CHUNK_S), jnp.int32),
        ]
        + [pltpu.VMEM((CHUNK_S, D), jnp.float32)] * NBUF
        + [pltpu.VMEM_SHARED((N_PAD, D), jnp.float32)]
        + [pltpu.SemaphoreType.DMA] * (2 * NBUF),
    )
    def k(g_hbm, src_hbm, dst_hbm, zero_hbm, out_hbm, src_v, dst_v, *rest):
        rows = rest[:NBUF]
        acc_sh = rest[NBUF]
        gsem = rest[NBUF + 1:NBUF + 1 + NBUF]
        ssem = rest[NBUF + 1 + NBUF:]
        c = lax.axis_index("c")
        s = lax.axis_index("s")
        wid = c * NS + s
        for kk in range(ROWS_PER_TILE // CHUNK):
            pltpu.sync_copy(
                zero_hbm, acc_sh.at[pl.ds(s * ROWS_PER_TILE + kk * CHUNK, CHUNK)])
        plsc.subcore_barrier()

        for h in range(N_IDX_LOADS):
            base = wid * CHUNKS_PER_TILE_S + h * IDX_HALF_S
            pltpu.sync_copy(src_hbm.at[pl.ds(base, IDX_HALF_S)], src_v)
            pltpu.sync_copy(dst_hbm.at[pl.ds(base, IDX_HALF_S)], dst_v)
            for b in range(NBUF):
                pltpu.async_copy(g_hbm.at[src_v.at[b]], rows[b], gsem[b])

            def body(t, carry):
                j = NBUF * t
                handles = []
                for b in range(NBUF):
                    pltpu.make_async_copy(
                        g_hbm.at[src_v.at[0]], rows[b], gsem[b]).wait()
                    handles.append(pltpu.async_copy(
                        rows[b], acc_sh.at[dst_v.at[j + b]], ssem[b], add=True))
                for b in range(NBUF):
                    jb = jnp.minimum(j + b + NBUF, IDX_HALF_S - 1)
                    handles[b].wait()
                    pltpu.async_copy(g_hbm.at[src_v.at[jb]], rows[b], gsem[b])
                return carry

            lax.fori_loop(0, IDX_HALF_S // NBUF, body, 0)
            for b in range(NBUF):
                pltpu.make_async_copy(
                    g_hbm.at[src_v.at[0]], rows[b], gsem[b]).wait()
        plsc.subcore_barrier()
        pltpu.sync_copy(
            acc_sh.at[pl.ds(s * ROWS_PER_TILE, ROWS_PER_TILE)],
            out_hbm.at[c, pl.ds(s * ROWS_PER_TILE, ROWS_PER_TILE)])

    return k(g, src_idx, dst_idx, zeros_rows)


_BLK = 1024
_GRID = 10


def _dis_block(dp):
    deg = dp[0, :, 0:1] + dp[1, :, 0:1] + 1.0
    return lax.rsqrt(deg)


def _tc_lin(x, W_lin, b_lin, W1, degp):
    def body(x_ref, wl_ref, bl_ref, w1_ref, dp_ref, out_ref):
        h = jnp.maximum(
            jnp.dot(x_ref[...], wl_ref[...], preferred_element_type=jnp.float32)
            + bl_ref[...], 0.0)
        dis = _dis_block(dp_ref[...])
        out_ref[...] = jnp.dot(
            h, w1_ref[...], preferred_element_type=jnp.float32) * dis

    return pl.pallas_call(
        body,
        grid=(_GRID,),
        in_specs=[
            pl.BlockSpec((_BLK, D), lambda i: (i, 0)),
            pl.BlockSpec((D, D), lambda i: (0, 0)),
            pl.BlockSpec((1, D), lambda i: (0, 0)),
            pl.BlockSpec((D, D), lambda i: (0, 0)),
            pl.BlockSpec((NC, _BLK, D), lambda i: (0, i, 0)),
        ],
        out_specs=pl.BlockSpec((_BLK, D), lambda i: (i, 0)),
        out_shape=jax.ShapeDtypeStruct((N_NODES, D), jnp.float32),
    )(x, W_lin, b_lin.reshape(1, D), W1, degp)


def _tc_mid(g1, S1, degp, b1, W2):
    def body(g_ref, s_ref, dp_ref, b_ref, w2_ref, out_ref):
        dis = _dis_block(dp_ref[...])
        ssum = s_ref[0] + s_ref[1]
        h1 = jnp.maximum(dis * (g_ref[...] + ssum) + b_ref[...], 0.0)
        out_ref[...] = jnp.dot(
            h1, w2_ref[...], preferred_element_type=jnp.float32) * dis

    return pl.pallas_call(
        body,
        grid=(_GRID,),
        in_specs=[
            pl.BlockSpec((_BLK, D), lambda i: (i, 0)),
            pl.BlockSpec((NC, _BLK, D), lambda i: (0, i, 0)),
            pl.BlockSpec((NC, _BLK, D), lambda i: (0, i, 0)),
            pl.BlockSpec((1, D), lambda i: (0, 0)),
            pl.BlockSpec((D, D), lambda i: (0, 0)),
        ],
        out_specs=pl.BlockSpec((_BLK, D), lambda i: (i, 0)),
        out_shape=jax.ShapeDtypeStruct((N_NODES, D), jnp.float32),
    )(g1, S1, degp, b1.reshape(1, D), W2)


def _tc_out(g2, S2, degp, b2):
    def body(g_ref, s_ref, dp_ref, b_ref, out_ref):
        dis = _dis_block(dp_ref[...])
        ssum = s_ref[0] + s_ref[1]
        out_ref[...] = jnp.maximum(dis * (g_ref[...] + ssum) + b_ref[...], 0.0)

    return pl.pallas_call(
        body,
        grid=(_GRID,),
        in_specs=[
            pl.BlockSpec((_BLK, D), lambda i: (i, 0)),
            pl.BlockSpec((NC, _BLK, D), lambda i: (0, i, 0)),
            pl.BlockSpec((NC, _BLK, D), lambda i: (0, i, 0)),
            pl.BlockSpec((1, D), lambda i: (0, 0)),
        ],
        out_specs=pl.BlockSpec((_BLK, D), lambda i: (i, 0)),
        out_shape=jax.ShapeDtypeStruct((N_NODES, D), jnp.float32),
    )(g2, S2, degp, b2.reshape(1, D))


def kernel(x, edge_index, W_lin, b_lin, W1, b1, W2, b2):
    src = edge_index[0].astype(jnp.int32)
    dst = edge_index[1].astype(jnp.int32)
    pad = E_PAD - N_EDGES
    ar = jnp.arange(pad, dtype=jnp.int32)
    src_p = jnp.concatenate(
        [src, ar % N_NODES]).reshape(N_CHUNKS, CHUNK)
    dst_p = jnp.concatenate(
        [dst, N_NODES + ar % (N_PAD - N_NODES)]).reshape(N_CHUNKS, CHUNK)

    onesD = jnp.ones((CHUNK, D), jnp.float32)
    zerosD = jnp.zeros((CHUNK, D), jnp.float32)

    src_s = src_p.reshape(N_CHUNKS_S, CHUNK_S)
    dst_s = dst_p.reshape(N_CHUNKS_S, CHUNK_S)

    degp = _sc_degree(dst_p, onesD, zerosD)
    g1 = _tc_lin(x, W_lin, b_lin, W1, degp)
    S1 = _sc_scatter(g1, src_s, dst_s, zerosD)
    g2 = _tc_mid(g1, S1, degp, b1, W2)
    S2 = _sc_scatter(g2, src_s, dst_s, zerosD)
    return _tc_out(g2, S2, degp, b2)


# speedup vs baseline: 1.1317x; 1.1317x over previous
import functools

import jax
import jax.numpy as jnp
from jax import lax
from jax.experimental import pallas as pl
from jax.experimental.pallas import tpu as pltpu
from jax.experimental.pallas import tpu_sc as plsc

N_NODES = 10000
N_EDGES = 320000
D = 128

NC = 2
NS = 16
NW = NC * NS

CHUNK = 128
CHUNKS_PER_TILE = 80
IDX_HALF = CHUNKS_PER_TILE // 2
E_PAD = NW * CHUNKS_PER_TILE * CHUNK
N_CHUNKS = E_PAD // CHUNK
N_PAD = 10240
ROWS_PER_TILE = N_PAD // NS

NBUF = 4
CHUNK_S = 64
CHUNKS_PER_TILE_S = E_PAD // (NW * CHUNK_S)
N_IDX_LOADS = 4
IDX_HALF_S = CHUNKS_PER_TILE_S // N_IDX_LOADS
N_CHUNKS_S = E_PAD // CHUNK_S


def _mesh():
    return plsc.VectorSubcoreMesh(
        core_axis_name="c", subcore_axis_name="s",
        num_cores=NC, num_subcores=NS)


def _sc_degree(dst_idx, ones_rows, zeros_rows):
    @functools.partial(
        pl.kernel,
        out_type=jax.ShapeDtypeStruct((NC, N_PAD, D), jnp.float32),
        mesh=_mesh(),
        scratch_types=[
            pltpu.VMEM((CHUNKS_PER_TILE, CHUNK), jnp.int32),
            pltpu.VMEM((CHUNK, D), jnp.float32),
            pltpu.VMEM_SHARED((N_PAD, D), jnp.float32),
            pltpu.SemaphoreType.DMA,
            pltpu.SemaphoreType.DMA,
        ],
    )
    def k(dst_hbm, ones_hbm, zero_hbm, out_hbm, dst_v, ones_v, acc_sh,
          sem0, sem1):
        c = lax.axis_index("c")
        s = lax.axis_index("s")
        wid = c * NS + s
        pltpu.sync_copy(zero_hbm, acc_sh.at[pl.ds(s * ROWS_PER_TILE, ROWS_PER_TILE)])
        pltpu.sync_copy(dst_hbm.at[pl.ds(wid * CHUNKS_PER_TILE, CHUNKS_PER_TILE)], dst_v)
        pltpu.sync_copy(ones_hbm, ones_v)
        plsc.subcore_barrier()

        h0 = pltpu.async_copy(ones_v, acc_sh.at[dst_v.at[0]], sem0, add=True)
        h1 = pltpu.async_copy(ones_v, acc_sh.at[dst_v.at[1]], sem1, add=True)

        def body(t, hs):
            j = 2 * t + 2
            hs[0].wait()
            n0 = pltpu.async_copy(ones_v, acc_sh.at[dst_v.at[j]], sem0, add=True)
            hs[1].wait()
            n1 = pltpu.async_copy(ones_v, acc_sh.at[dst_v.at[j + 1]], sem1,
                                  add=True)
            return [n0, n1]

        hs = [h0, h1]
        for t2 in range((CHUNKS_PER_TILE - 2) // 2):
            hs = body(t2, hs)
        hs[0].wait()
        hs[1].wait()
        plsc.subcore_barrier()
        pltpu.sync_copy(
            acc_sh.at[pl.ds(s * ROWS_PER_TILE, ROWS_PER_TILE)],
            out_hbm.at[c, pl.ds(s * ROWS_PER_TILE, ROWS_PER_TILE)])

    return k(dst_idx, ones_rows, zeros_rows)


def _sc_scatter(g, src_idx, dst_idx, zeros_rows):
    @functools.partial(
        pl.kernel,
        out_type=jax.ShapeDtypeStruct((NC, N_PAD, D), jnp.float32),
        mesh=_mesh(),
        scratch_types=[
            pltpu.VMEM((IDX_HALF_S, CHUNK_S), jnp.int32),
            pltpu.VMEM((IDX_HALF_S, CHUNK_S), jnp.int32),
        ]
        + [pltpu.VMEM((CHUNK_S, D), jnp.float32)] * NBUF
        + [pltpu.VMEM_SHARED((N_PAD, D), jnp.float32)]
        + [pltpu.SemaphoreType.DMA] * (2 * NBUF),
    )
    def k(g_hbm, src_hbm, dst_hbm, zero_hbm, out_hbm, src_v, dst_v, *rest):
        rows = rest[:NBUF]
        acc_sh = rest[NBUF]
        gsem = rest[NBUF + 1:NBUF + 1 + NBUF]
        ssem = rest[NBUF + 1 + NBUF:]
        c = lax.axis_index("c")
        s = lax.axis_index("s")
        wid = c * NS + s
        pltpu.sync_copy(zero_hbm, acc_sh.at[pl.ds(s * ROWS_PER_TILE, ROWS_PER_TILE)])
        plsc.subcore_barrier()

        for h in range(N_IDX_LOADS):
            base = wid * CHUNKS_PER_TILE_S + h * IDX_HALF_S
            pltpu.sync_copy(src_hbm.at[pl.ds(base, IDX_HALF_S)], src_v)
            pltpu.sync_copy(dst_hbm.at[pl.ds(base, IDX_HALF_S)], dst_v)
            for b in range(NBUF):
                pltpu.async_copy(g_hbm.at[src_v.at[b]], rows[b], gsem[b])

            def body(t, carry):
                j = NBUF * t
                handles = []
                for b in range(NBUF):
                    pltpu.make_async_copy(
                        g_hbm.at[src_v.at[0]], rows[b], gsem[b]).wait()
                    handles.append(pltpu.async_copy(
                        rows[b], acc_sh.at[dst_v.at[j + b]], ssem[b], add=True))
                for b in range(NBUF):
                    jb = jnp.minimum(j + b + NBUF, IDX_HALF_S - 1)
                    handles[b].wait()
                    pltpu.async_copy(g_hbm.at[src_v.at[jb]], rows[b], gsem[b])
                return carry

            lax.fori_loop(0, IDX_HALF_S // NBUF, body, 0)
            for b in range(NBUF):
                pltpu.make_async_copy(
                    g_hbm.at[src_v.at[0]], rows[b], gsem[b]).wait()
        plsc.subcore_barrier()
        pltpu.sync_copy(
            acc_sh.at[pl.ds(s * ROWS_PER_TILE, ROWS_PER_TILE)],
            out_hbm.at[c, pl.ds(s * ROWS_PER_TILE, ROWS_PER_TILE)])

    return k(g, src_idx, dst_idx, zeros_rows)


_BLK = 1024
_GRID = 10


def _dis_block(dp):
    deg = dp[0, :, 0:1] + dp[1, :, 0:1] + 1.0
    return lax.rsqrt(deg)


def _tc_lin(x, W_lin, b_lin, W1, degp):
    def body(x_ref, wl_ref, bl_ref, w1_ref, dp_ref, out_ref):
        h = jnp.maximum(
            jnp.dot(x_ref[...], wl_ref[...], preferred_element_type=jnp.float32)
            + bl_ref[...], 0.0)
        dis = _dis_block(dp_ref[...])
        out_ref[...] = jnp.dot(
            h, w1_ref[...], preferred_element_type=jnp.float32) * dis

    return pl.pallas_call(
        body,
        grid=(_GRID,),
        in_specs=[
            pl.BlockSpec((_BLK, D), lambda i: (i, 0)),
            pl.BlockSpec((D, D), lambda i: (0, 0)),
            pl.BlockSpec((1, D), lambda i: (0, 0)),
            pl.BlockSpec((D, D), lambda i: (0, 0)),
            pl.BlockSpec((NC, _BLK, D), lambda i: (0, i, 0)),
        ],
        out_specs=pl.BlockSpec((_BLK, D), lambda i: (i, 0)),
        out_shape=jax.ShapeDtypeStruct((N_NODES, D), jnp.float32),
    )(x, W_lin, b_lin.reshape(1, D), W1, degp)


def _tc_mid(g1, S1, degp, b1, W2):
    def body(g_ref, s_ref, dp_ref, b_ref, w2_ref, out_ref):
        dis = _dis_block(dp_ref[...])
        ssum = s_ref[0] + s_ref[1]
        h1 = jnp.maximum(dis * (g_ref[...] + ssum) + b_ref[...], 0.0)
        out_ref[...] = jnp.dot(
            h1, w2_ref[...], preferred_element_type=jnp.float32) * dis

    return pl.pallas_call(
        body,
        grid=(_GRID,),
        in_specs=[
            pl.BlockSpec((_BLK, D), lambda i: (i, 0)),
            pl.BlockSpec((NC, _BLK, D), lambda i: (0, i, 0)),
            pl.BlockSpec((NC, _BLK, D), lambda i: (0, i, 0)),
            pl.BlockSpec((1, D), lambda i: (0, 0)),
            pl.BlockSpec((D, D), lambda i: (0, 0)),
        ],
        out_specs=pl.BlockSpec((_BLK, D), lambda i: (i, 0)),
        out_shape=jax.ShapeDtypeStruct((N_NODES, D), jnp.float32),
    )(g1, S1, degp, b1.reshape(1, D), W2)


def _tc_out(g2, S2, degp, b2):
    def body(g_ref, s_ref, dp_ref, b_ref, out_ref):
        dis = _dis_block(dp_ref[...])
        ssum = s_ref[0] + s_ref[1]
        out_ref[...] = jnp.maximum(dis * (g_ref[...] + ssum) + b_ref[...], 0.0)

    return pl.pallas_call(
        body,
        grid=(_GRID,),
        in_specs=[
            pl.BlockSpec((_BLK, D), lambda i: (i, 0)),
            pl.BlockSpec((NC, _BLK, D), lambda i: (0, i, 0)),
            pl.BlockSpec((NC, _BLK, D), lambda i: (0, i, 0)),
            pl.BlockSpec((1, D), lambda i: (0, 0)),
        ],
        out_specs=pl.BlockSpec((_BLK, D), lambda i: (i, 0)),
        out_shape=jax.ShapeDtypeStruct((N_NODES, D), jnp.float32),
    )(g2, S2, degp, b2.reshape(1, D))


def kernel(x, edge_index, W_lin, b_lin, W1, b1, W2, b2):
    src = edge_index[0].astype(jnp.int32)
    dst = edge_index[1].astype(jnp.int32)
    pad = E_PAD - N_EDGES
    ar = jnp.arange(pad, dtype=jnp.int32)
    src_p = jnp.concatenate(
        [src, ar % N_NODES]).reshape(N_CHUNKS, CHUNK)
    dst_p = jnp.concatenate(
        [dst, N_NODES + ar % (N_PAD - N_NODES)]).reshape(N_CHUNKS, CHUNK)

    onesD = jnp.ones((CHUNK, D), jnp.float32)
    zerosD = jnp.zeros((ROWS_PER_TILE, D), jnp.float32)

    src_s = src_p.reshape(N_CHUNKS_S, CHUNK_S)
    dst_s = dst_p.reshape(N_CHUNKS_S, CHUNK_S)

    degp = _sc_degree(dst_p, onesD, zerosD)
    g1 = _tc_lin(x, W_lin, b_lin, W1, degp)
    S1 = _sc_scatter(g1, src_s, dst_s, zerosD)
    g2 = _tc_mid(g1, S1, degp, b1, W2)
    S2 = _sc_scatter(g2, src_s, dst_s, zerosD)
    return _tc_out(g2, S2, degp, b2)


# speedup vs baseline: 1.1344x; 1.0024x over previous
import functools

import jax
import jax.numpy as jnp
from jax import lax
from jax.experimental import pallas as pl
from jax.experimental.pallas import tpu as pltpu
from jax.experimental.pallas import tpu_sc as plsc

N_NODES = 10000
N_EDGES = 320000
D = 128

NC = 2
NS = 16
NW = NC * NS

CHUNK = 128
CHUNKS_PER_TILE = 80
IDX_HALF = CHUNKS_PER_TILE // 2
E_PAD = NW * CHUNKS_PER_TILE * CHUNK
N_CHUNKS = E_PAD // CHUNK
N_PAD = 10240
ROWS_PER_TILE = N_PAD // NS

NBUF = 4
CHUNK_S = 64
CHUNKS_PER_TILE_S = E_PAD // (NW * CHUNK_S)
N_IDX_LOADS = 4
IDX_HALF_S = CHUNKS_PER_TILE_S // N_IDX_LOADS
N_CHUNKS_S = E_PAD // CHUNK_S


def _mesh():
    return plsc.VectorSubcoreMesh(
        core_axis_name="c", subcore_axis_name="s",
        num_cores=NC, num_subcores=NS)


def _sc_degree(dst_idx, ones_rows, zeros_rows):
    @functools.partial(
        pl.kernel,
        out_type=jax.ShapeDtypeStruct((NC, N_PAD, D), jnp.float32),
        mesh=_mesh(),
        scratch_types=[
            pltpu.VMEM((CHUNKS_PER_TILE, CHUNK), jnp.int32),
            pltpu.VMEM((CHUNK, D), jnp.float32),
            pltpu.VMEM_SHARED((N_PAD, D), jnp.float32),
        ]
        + [pltpu.SemaphoreType.DMA] * 4,
    )
    def k(dst_hbm, ones_hbm, zero_hbm, out_hbm, dst_v, ones_v, acc_sh,
          *sems):
        c = lax.axis_index("c")
        s = lax.axis_index("s")
        wid = c * NS + s
        pltpu.sync_copy(zero_hbm, acc_sh.at[pl.ds(s * ROWS_PER_TILE, ROWS_PER_TILE)])
        pltpu.sync_copy(dst_hbm.at[pl.ds(wid * CHUNKS_PER_TILE, CHUNKS_PER_TILE)], dst_v)
        pltpu.sync_copy(ones_hbm, ones_v)
        plsc.subcore_barrier()

        DW = 4
        hs = [pltpu.async_copy(ones_v, acc_sh.at[dst_v.at[b]], sems[b],
                               add=True) for b in range(DW)]
        for t2 in range((CHUNKS_PER_TILE - DW) // DW):
            j = DW * t2 + DW
            for b in range(DW):
                hs[b].wait()
                hs[b] = pltpu.async_copy(
                    ones_v, acc_sh.at[dst_v.at[j + b]], sems[b], add=True)
        for b in range(DW):
            hs[b].wait()
        plsc.subcore_barrier()
        pltpu.sync_copy(
            acc_sh.at[pl.ds(s * ROWS_PER_TILE, ROWS_PER_TILE)],
            out_hbm.at[c, pl.ds(s * ROWS_PER_TILE, ROWS_PER_TILE)])

    return k(dst_idx, ones_rows, zeros_rows)


def _sc_scatter(g, src_idx, dst_idx, zeros_rows):
    @functools.partial(
        pl.kernel,
        out_type=jax.ShapeDtypeStruct((NC, N_PAD, D), jnp.float32),
        mesh=_mesh(),
        scratch_types=[
            pltpu.VMEM((IDX_HALF_S, CHUNK_S), jnp.int32),
            pltpu.VMEM((IDX_HALF_S, CHUNK_S), jnp.int32),
        ]
        + [pltpu.VMEM((CHUNK_S, D), jnp.float32)] * NBUF
        + [pltpu.VMEM_SHARED((N_PAD, D), jnp.float32)]
        + [pltpu.SemaphoreType.DMA] * (2 * NBUF),
    )
    def k(g_hbm, src_hbm, dst_hbm, zero_hbm, out_hbm, src_v, dst_v, *rest):
        rows = rest[:NBUF]
        acc_sh = rest[NBUF]
        gsem = rest[NBUF + 1:NBUF + 1 + NBUF]
        ssem = rest[NBUF + 1 + NBUF:]
        c = lax.axis_index("c")
        s = lax.axis_index("s")
        wid = c * NS + s
        pltpu.sync_copy(zero_hbm, acc_sh.at[pl.ds(s * ROWS_PER_TILE, ROWS_PER_TILE)])
        plsc.subcore_barrier()

        for h in range(N_IDX_LOADS):
            base = wid * CHUNKS_PER_TILE_S + h * IDX_HALF_S
            pltpu.sync_copy(src_hbm.at[pl.ds(base, IDX_HALF_S)], src_v)
            pltpu.sync_copy(dst_hbm.at[pl.ds(base, IDX_HALF_S)], dst_v)
            for b in range(NBUF):
                pltpu.async_copy(g_hbm.at[src_v.at[b]], rows[b], gsem[b])

            def body(t, carry):
                j = NBUF * t
                handles = []
                for b in range(NBUF):
                    pltpu.make_async_copy(
                        g_hbm.at[src_v.at[0]], rows[b], gsem[b]).wait()
                    handles.append(pltpu.async_copy(
                        rows[b], acc_sh.at[dst_v.at[j + b]], ssem[b], add=True))
                for b in range(NBUF):
                    jb = jnp.minimum(j + b + NBUF, IDX_HALF_S - 1)
                    handles[b].wait()
                    pltpu.async_copy(g_hbm.at[src_v.at[jb]], rows[b], gsem[b])
                return carry

            lax.fori_loop(0, IDX_HALF_S // NBUF, body, 0)
            for b in range(NBUF):
                pltpu.make_async_copy(
                    g_hbm.at[src_v.at[0]], rows[b], gsem[b]).wait()
        plsc.subcore_barrier()
        pltpu.sync_copy(
            acc_sh.at[pl.ds(s * ROWS_PER_TILE, ROWS_PER_TILE)],
            out_hbm.at[c, pl.ds(s * ROWS_PER_TILE, ROWS_PER_TILE)])

    return k(g, src_idx, dst_idx, zeros_rows)


_BLK = 1024
_GRID = 10


def _dis_block(dp):
    deg = dp[0, :, 0:1] + dp[1, :, 0:1] + 1.0
    return lax.rsqrt(deg)


def _tc_lin(x, W_lin, b_lin, W1, degp):
    def body(x_ref, wl_ref, bl_ref, w1_ref, dp_ref, out_ref):
        h = jnp.maximum(
            jnp.dot(x_ref[...], wl_ref[...], preferred_element_type=jnp.float32)
            + bl_ref[...], 0.0)
        dis = _dis_block(dp_ref[...])
        out_ref[...] = jnp.dot(
            h, w1_ref[...], preferred_element_type=jnp.float32) * dis

    return pl.pallas_call(
        body,
        grid=(_GRID,),
        in_specs=[
            pl.BlockSpec((_BLK, D), lambda i: (i, 0)),
            pl.BlockSpec((D, D), lambda i: (0, 0)),
            pl.BlockSpec((1, D), lambda i: (0, 0)),
            pl.BlockSpec((D, D), lambda i: (0, 0)),
            pl.BlockSpec((NC, _BLK, D), lambda i: (0, i, 0)),
        ],
        out_specs=pl.BlockSpec((_BLK, D), lambda i: (i, 0)),
        out_shape=jax.ShapeDtypeStruct((N_NODES, D), jnp.float32),
    )(x, W_lin, b_lin.reshape(1, D), W1, degp)


def _tc_mid(g1, S1, degp, b1, W2):
    def body(g_ref, s_ref, dp_ref, b_ref, w2_ref, out_ref):
        dis = _dis_block(dp_ref[...])
        ssum = s_ref[0] + s_ref[1]
        h1 = jnp.maximum(dis * (g_ref[...] + ssum) + b_ref[...], 0.0)
        out_ref[...] = jnp.dot(
            h1, w2_ref[...], preferred_element_type=jnp.float32) * dis

    return pl.pallas_call(
        body,
        grid=(_GRID,),
        in_specs=[
            pl.BlockSpec((_BLK, D), lambda i: (i, 0)),
            pl.BlockSpec((NC, _BLK, D), lambda i: (0, i, 0)),
            pl.BlockSpec((NC, _BLK, D), lambda i: (0, i, 0)),
            pl.BlockSpec((1, D), lambda i: (0, 0)),
            pl.BlockSpec((D, D), lambda i: (0, 0)),
        ],
        out_specs=pl.BlockSpec((_BLK, D), lambda i: (i, 0)),
        out_shape=jax.ShapeDtypeStruct((N_NODES, D), jnp.float32),
    )(g1, S1, degp, b1.reshape(1, D), W2)


def _tc_out(g2, S2, degp, b2):
    def body(g_ref, s_ref, dp_ref, b_ref, out_ref):
        dis = _dis_block(dp_ref[...])
        ssum = s_ref[0] + s_ref[1]
        out_ref[...] = jnp.maximum(dis * (g_ref[...] + ssum) + b_ref[...], 0.0)

    return pl.pallas_call(
        body,
        grid=(_GRID,),
        in_specs=[
            pl.BlockSpec((_BLK, D), lambda i: (i, 0)),
            pl.BlockSpec((NC, _BLK, D), lambda i: (0, i, 0)),
            pl.BlockSpec((NC, _BLK, D), lambda i: (0, i, 0)),
            pl.BlockSpec((1, D), lambda i: (0, 0)),
        ],
        out_specs=pl.BlockSpec((_BLK, D), lambda i: (i, 0)),
        out_shape=jax.ShapeDtypeStruct((N_NODES, D), jnp.float32),
    )(g2, S2, degp, b2.reshape(1, D))


def kernel(x, edge_index, W_lin, b_lin, W1, b1, W2, b2):
    src = edge_index[0].astype(jnp.int32)
    dst = edge_index[1].astype(jnp.int32)
    pad = E_PAD - N_EDGES
    ar = jnp.arange(pad, dtype=jnp.int32)
    src_p = jnp.concatenate(
        [src, ar % N_NODES]).reshape(N_CHUNKS, CHUNK)
    dst_p = jnp.concatenate(
        [dst, N_NODES + ar % (N_PAD - N_NODES)]).reshape(N_CHUNKS, CHUNK)

    onesD = jnp.ones((CHUNK, D), jnp.float32)
    zerosD = jnp.zeros((ROWS_PER_TILE, D), jnp.float32)

    src_s = src_p.reshape(N_CHUNKS_S, CHUNK_S)
    dst_s = dst_p.reshape(N_CHUNKS_S, CHUNK_S)

    degp = _sc_degree(dst_p, onesD, zerosD)
    g1 = _tc_lin(x, W_lin, b_lin, W1, degp)
    S1 = _sc_scatter(g1, src_s, dst_s, zerosD)
    g2 = _tc_mid(g1, S1, degp, b1, W2)
    S2 = _sc_scatter(g2, src_s, dst_s, zerosD)
    return _tc_out(g2, S2, degp, b2)
